# baseline (speedup 1.0000x reference)
import functools
import jax
import jax.numpy as jnp
from jax import lax
from jax.experimental import pallas as pl
from jax.experimental.pallas import tpu as pltpu
from jax.experimental.pallas import tpu_sc as plsc

_N, _U, _I, _E = 1024, 26, 26, 64
_NW = 32
_BPW = _N // _NW
_ROWS = _U * _I


def _sc_body(xu_hbm, xi_hbm, out_hbm, xu_v, xi_v, buf, sem):
    nc = 2
    wid = lax.axis_index("s") * nc + lax.axis_index("c")
    b0 = wid * _BPW

    pltpu.sync_copy(xu_hbm.at[b0], xu_v)
    pltpu.sync_copy(xi_hbm.at[b0], xi_v)

    def batch_body(t, _):
        b = b0 + t

        def ibody(i, _):
            base = 26 * i
            u = [xu_v[i, pl.ds(16 * k, 16)] for k in range(4)]
            for j in range(_I):
                for k in range(4):
                    buf[base + j, pl.ds(16 * k, 16)] = u[k]
                for k in range(4):
                    buf[base + j, pl.ds(64 + 16 * k, 16)] = xi_v[j, pl.ds(16 * k, 16)]
            return None

        lax.fori_loop(0, _U, ibody, None)

        copy = pltpu.async_copy(buf, out_hbm.at[b], sem)
        @pl.when(t < _BPW - 1)
        def _stage():
            pltpu.sync_copy(xu_hbm.at[b + 1], xu_v)
            pltpu.sync_copy(xi_hbm.at[b + 1], xi_v)

        copy.wait()
        return None

    lax.fori_loop(0, _BPW, batch_body, None)


@jax.jit
def kernel(x_user, x_item):
    n, u, e = x_user.shape
    i = x_item.shape[1]
    mesh = plsc.VectorSubcoreMesh(core_axis_name="c", subcore_axis_name="s")
    f = functools.partial(
        pl.kernel,
        mesh=mesh,
        out_type=jax.ShapeDtypeStruct((n, u * i, 2 * e), jnp.float32),
        scratch_types=[
            pltpu.VMEM((u, e), jnp.float32),
            pltpu.VMEM((i, e), jnp.float32),
            pltpu.VMEM((u * i, 2 * e), jnp.float32),
            pltpu.SemaphoreType.DMA,
        ],
    )(_sc_body)
    return f(x_user, x_item)


# speedup vs baseline: 1.4113x; 1.4113x over previous
import functools
import jax
import jax.numpy as jnp
from jax import lax
from jax.experimental import pallas as pl
from jax.experimental.pallas import tpu as pltpu
from jax.experimental.pallas import tpu_sc as plsc

_N, _U, _I, _E = 1024, 26, 26, 64
_NW = 32
_BPW = _N // _NW
_ROWS = _U * _I


def _sc_body(xu_hbm, xi_hbm, out_hbm, xu_v, xi_v, buf, sem):
    nc = 2
    wid = lax.axis_index("s") * nc + lax.axis_index("c")
    b0 = wid * _BPW

    pltpu.sync_copy(xu_hbm.at[b0], xu_v)
    pltpu.sync_copy(xi_hbm.at[b0], xi_v)

    def batch_body(t, _):
        b = b0 + t
        for half in range(2):
            jbase = 13 * half
            items = [
                xi_v[jbase + jj, pl.ds(16 * k, 16)]
                for jj in range(13)
                for k in range(4)
            ]

            def ibody(i, _, jbase=jbase, items=items):
                base = 26 * i + jbase
                u = [xu_v[i, pl.ds(16 * k, 16)] for k in range(4)]
                for jj in range(13):
                    for k in range(4):
                        buf[base + jj, pl.ds(16 * k, 16)] = u[k]
                    for k in range(4):
                        buf[base + jj, pl.ds(64 + 16 * k, 16)] = items[4 * jj + k]
                return None

            lax.fori_loop(0, _U, ibody, None)

        copy = pltpu.async_copy(buf, out_hbm.at[b], sem)
        @pl.when(t < _BPW - 1)
        def _stage():
            pltpu.sync_copy(xu_hbm.at[b + 1], xu_v)
            pltpu.sync_copy(xi_hbm.at[b + 1], xi_v)

        copy.wait()
        return None

    lax.fori_loop(0, _BPW, batch_body, None)


@jax.jit
def kernel(x_user, x_item):
    n, u, e = x_user.shape
    i = x_item.shape[1]
    mesh = plsc.VectorSubcoreMesh(core_axis_name="c", subcore_axis_name="s")
    f = functools.partial(
        pl.kernel,
        mesh=mesh,
        out_type=jax.ShapeDtypeStruct((n, u * i, 2 * e), jnp.float32),
        scratch_types=[
            pltpu.VMEM((u, e), jnp.float32),
            pltpu.VMEM((i, e), jnp.float32),
            pltpu.VMEM((u * i, 2 * e), jnp.float32),
            pltpu.SemaphoreType.DMA,
        ],
    )(_sc_body)
    return f(x_user, x_item)
